# baseline (speedup 1.0000x reference)
import functools

import jax
import jax.numpy as jnp
from jax import lax
from jax.experimental import pallas as pl
from jax.experimental.pallas import tpu as pltpu
from jax.experimental.pallas import tpu_sc as plsc

_BS = 128
_COPY = 64
_NW = 32
_RPW = _BS // _NW
_WIN = 80


def _sc_gather(rpi, table, start):
    mesh = plsc.VectorSubcoreMesh(core_axis_name="c", subcore_axis_name="s")

    @functools.partial(
        pl.kernel,
        out_type=jax.ShapeDtypeStruct((_BS * _COPY,), jnp.int32),
        mesh=mesh,
        compiler_params=pltpu.CompilerParams(
            use_tc_tiling_on_sc=False, needs_layout_passes=False
        ),
        scratch_types=[
            pltpu.VMEM((_BS + 16,), jnp.int32),
            pltpu.VMEM((_BS + 16,), jnp.int32),
            pltpu.VMEM((_RPW * _WIN,), jnp.int32),
            pltpu.VMEM((_RPW * _COPY,), jnp.int32),
            pltpu.SemaphoreType.DMA,
        ],
    )
    def k(rpi_hbm, table_hbm, start_hbm, out_hbm, rpi_v, start_v, buf_v, out_v, sem):
        wid = lax.axis_index("s") * 2 + lax.axis_index("c")
        base = wid * _RPW
        pltpu.sync_copy(rpi_hbm, rpi_v.at[pl.ds(0, _BS)])
        pltpu.sync_copy(start_hbm, start_v.at[pl.ds(0, _BS)])
        rows16 = rpi_v[pl.ds(base, 16)]
        starts16 = start_v[pl.ds(base, 16)]
        copies = []
        for q in range(_RPW):
            sa = pl.multiple_of(starts16[q] & ~jnp.int32(15), 16)
            copies.append(
                pltpu.async_copy(
                    table_hbm.at[rows16[q], pl.ds(sa, _WIN)],
                    buf_v.at[pl.ds(q * _WIN, _WIN)],
                    sem,
                )
            )
        for c in copies:
            c.wait()
        lanes = lax.iota(jnp.int32, 16)
        for q in range(_RPW):
            phv = jnp.full((16,), q * _WIN, jnp.int32) + (starts16[q] & jnp.int32(15))
            for j0 in range(0, _COPY, 16):
                idx = phv + (lanes + j0)
                out_v[pl.ds(q * _COPY + j0, 16)] = plsc.load_gather(buf_v, [idx])
        pltpu.sync_copy(out_v, out_hbm.at[pl.ds(wid * (_RPW * _COPY), _RPW * _COPY)])

    return k(rpi, table, start)


def kernel(req_pool_indices, req_to_token, seq_lens, topk, speculative_num_steps):
    dep = jnp.asarray(topk * speculative_num_steps - _COPY).astype(seq_lens.dtype)
    start = (seq_lens + dep).astype(jnp.int32)
    rpi = req_pool_indices.astype(jnp.int32)
    return _sc_gather(rpi, req_to_token.astype(jnp.int32), start)


# speedup vs baseline: 3.1768x; 3.1768x over previous
import functools

import jax
import jax.numpy as jnp
from jax import lax
from jax.experimental import pallas as pl
from jax.experimental.pallas import tpu as pltpu
from jax.experimental.pallas import tpu_sc as plsc

_BS = 128
_COPY = 64
_NW = 32
_RPW = _BS // _NW
_POOL_LEN = 32768


def _sc_gather(rpi, table, start):
    mesh = plsc.VectorSubcoreMesh(core_axis_name="c", subcore_axis_name="s")

    @functools.partial(
        pl.kernel,
        out_type=jax.ShapeDtypeStruct((_BS * _COPY,), jnp.int32),
        mesh=mesh,
        compiler_params=pltpu.CompilerParams(needs_layout_passes=False),
        scratch_types=[
            pltpu.VMEM((_BS + 16,), jnp.int32),
            pltpu.VMEM((_BS + 16,), jnp.int32),
            pltpu.VMEM((_RPW * 2, 8, 128), jnp.int32),
            pltpu.VMEM((_RPW * _COPY,), jnp.int32),
            pltpu.SemaphoreType.DMA,
        ],
    )
    def k(rpi_hbm, table_hbm, start_hbm, out_hbm, rpi_v, start_v, buf_v, out_v, sem):
        wid = lax.axis_index("s") * 2 + lax.axis_index("c")
        base = wid * _RPW
        pltpu.sync_copy(rpi_hbm, rpi_v.at[pl.ds(0, _BS)])
        pltpu.sync_copy(start_hbm, start_v.at[pl.ds(0, _BS)])
        rows16 = rpi_v[pl.ds(base, 16)]
        starts16 = start_v[pl.ds(base, 16)]
        copies = []
        cas = []
        for q in range(_RPW):
            r8 = pl.multiple_of(rows16[q] & ~jnp.int32(7), 8)
            ca = jnp.minimum(
                starts16[q] & ~jnp.int32(127), jnp.int32(_POOL_LEN - 256)
            )
            ca = pl.multiple_of(ca, 128)
            cas.append(ca)
            for t in range(2):
                copies.append(
                    pltpu.async_copy(
                        table_hbm.at[pl.ds(r8, 8), pl.ds(ca + t * 128, 128)],
                        buf_v.at[q * 2 + t],
                        sem,
                    )
                )
        for c in copies:
            c.wait()
        lanes = lax.iota(jnp.int32, 16)
        for q in range(_RPW):
            rv = jnp.full((16,), rows16[q] & jnp.int32(7), jnp.int32)
            phv = jnp.full((16,), starts16[q] - cas[q], jnp.int32) + lanes
            for j0 in range(0, _COPY, 16):
                lc = phv + j0
                bv = (lc >> 7) + (q * 2)
                out_v[pl.ds(q * _COPY + j0, 16)] = plsc.load_gather(
                    buf_v, [bv, rv, lc & jnp.int32(127)]
                )
        pltpu.sync_copy(out_v, out_hbm.at[pl.ds(wid * (_RPW * _COPY), _RPW * _COPY)])

    return k(rpi, table, start)


def kernel(req_pool_indices, req_to_token, seq_lens, topk, speculative_num_steps):
    dep = jnp.asarray(topk * speculative_num_steps - _COPY).astype(seq_lens.dtype)
    start = (seq_lens + dep).astype(jnp.int32)
    rpi = req_pool_indices.astype(jnp.int32)
    return _sc_gather(rpi, req_to_token.astype(jnp.int32), start)


# speedup vs baseline: 3.2511x; 1.0234x over previous
import functools

import jax
import jax.numpy as jnp
from jax import lax
from jax.experimental import pallas as pl
from jax.experimental.pallas import tpu as pltpu
from jax.experimental.pallas import tpu_sc as plsc

_BS = 128
_COPY = 64
_NW = 32
_RPW = _BS // _NW
_POOL_LEN = 32768


def _sc_gather(rpi, table, start):
    mesh = plsc.VectorSubcoreMesh(core_axis_name="c", subcore_axis_name="s")

    @functools.partial(
        pl.kernel,
        out_type=jax.ShapeDtypeStruct((_BS * _COPY,), jnp.int32),
        mesh=mesh,
        compiler_params=pltpu.CompilerParams(
            needs_layout_passes=False, skip_device_barrier=True
        ),
        scratch_types=[
            pltpu.VMEM((_BS + 16,), jnp.int32),
            pltpu.VMEM((_BS + 16,), jnp.int32),
            pltpu.VMEM((_RPW * 2, 8, 128), jnp.int32),
            pltpu.VMEM((_RPW * _COPY,), jnp.int32),
            pltpu.SemaphoreType.DMA,
        ],
    )
    def k(rpi_hbm, table_hbm, start_hbm, out_hbm, rpi_v, start_v, buf_v, out_v, sem):
        wid = lax.axis_index("s") * 2 + lax.axis_index("c")
        base = wid * _RPW
        c1 = pltpu.async_copy(rpi_hbm, rpi_v.at[pl.ds(0, _BS)], sem)
        c2 = pltpu.async_copy(start_hbm, start_v.at[pl.ds(0, _BS)], sem)
        c1.wait()
        c2.wait()
        rows16 = rpi_v[pl.ds(base, 16)]
        starts16 = start_v[pl.ds(base, 16)]
        copies = []
        cas = []
        for q in range(_RPW):
            r8 = pl.multiple_of(rows16[q] & ~jnp.int32(7), 8)
            ca = jnp.minimum(
                starts16[q] & ~jnp.int32(127), jnp.int32(_POOL_LEN - 256)
            )
            ca = pl.multiple_of(ca, 128)
            cas.append(ca)
            for t in range(2):
                copies.append(
                    pltpu.async_copy(
                        table_hbm.at[pl.ds(r8, 8), pl.ds(ca + t * 128, 128)],
                        buf_v.at[q * 2 + t],
                        sem,
                    )
                )
        for c in copies:
            c.wait()
        lanes = lax.iota(jnp.int32, 16)
        for q in range(_RPW):
            rv = jnp.full((16,), rows16[q] & jnp.int32(7), jnp.int32)
            phv = jnp.full((16,), starts16[q] - cas[q], jnp.int32) + lanes
            for j0 in range(0, _COPY, 16):
                lc = phv + j0
                bv = (lc >> 7) + (q * 2)
                out_v[pl.ds(q * _COPY + j0, 16)] = plsc.load_gather(
                    buf_v, [bv, rv, lc & jnp.int32(127)]
                )
        pltpu.sync_copy(out_v, out_hbm.at[pl.ds(wid * (_RPW * _COPY), _RPW * _COPY)])

    return k(rpi, table, start)


def kernel(req_pool_indices, req_to_token, seq_lens, topk, speculative_num_steps):
    dep = jnp.asarray(topk * speculative_num_steps - _COPY).astype(seq_lens.dtype)
    start = (seq_lens + dep).astype(jnp.int32)
    rpi = req_pool_indices.astype(jnp.int32)
    return _sc_gather(rpi, req_to_token.astype(jnp.int32), start)


# speedup vs baseline: 3.2658x; 1.0045x over previous
import functools

import jax
import jax.numpy as jnp
from jax import lax
from jax.experimental import pallas as pl
from jax.experimental.pallas import tpu as pltpu
from jax.experimental.pallas import tpu_sc as plsc

_BS = 128
_COPY = 64
_NW = 32
_RPW = _BS // _NW
_POOL_LEN = 32768


def _sc_gather(rpi, table, seq, tk1, st1):
    mesh = plsc.VectorSubcoreMesh(core_axis_name="c", subcore_axis_name="s")

    @functools.partial(
        pl.kernel,
        out_type=jax.ShapeDtypeStruct((_BS * _COPY,), jnp.int32),
        mesh=mesh,
        compiler_params=pltpu.CompilerParams(
            needs_layout_passes=False, skip_device_barrier=True
        ),
        scratch_types=[
            pltpu.VMEM((_BS + 16,), jnp.int32),
            pltpu.VMEM((_BS + 16,), jnp.int32),
            pltpu.VMEM((16,), jnp.int32),
            pltpu.VMEM((_RPW * 2, 8, 128), jnp.int32),
            pltpu.VMEM((_RPW * _COPY,), jnp.int32),
            pltpu.SemaphoreType.DMA,
        ],
    )
    def k(rpi_hbm, table_hbm, seq_hbm, tk_hbm, st_hbm, out_hbm,
          rpi_v, seq_v, sc_v, buf_v, out_v, sem):
        wid = lax.axis_index("s") * 2 + lax.axis_index("c")
        base = wid * _RPW
        c1 = pltpu.async_copy(rpi_hbm, rpi_v.at[pl.ds(0, _BS)], sem)
        c2 = pltpu.async_copy(seq_hbm, seq_v.at[pl.ds(0, _BS)], sem)
        c3 = pltpu.async_copy(tk_hbm, sc_v.at[pl.ds(0, 1)], sem)
        c4 = pltpu.async_copy(st_hbm, sc_v.at[pl.ds(8, 1)], sem)
        c1.wait()
        c2.wait()
        c3.wait()
        c4.wait()
        scal = sc_v[pl.ds(0, 16)]
        dep = scal[0] * scal[8] - jnp.int32(_COPY)
        rows16 = rpi_v[pl.ds(base, 16)]
        starts16 = seq_v[pl.ds(base, 16)] + dep
        copies = []
        cas = []
        for q in range(_RPW):
            r8 = pl.multiple_of(rows16[q] & ~jnp.int32(7), 8)
            ca = jnp.minimum(
                starts16[q] & ~jnp.int32(127), jnp.int32(_POOL_LEN - 256)
            )
            ca = pl.multiple_of(ca, 128)
            cas.append(ca)
            for t in range(2):
                copies.append(
                    pltpu.async_copy(
                        table_hbm.at[pl.ds(r8, 8), pl.ds(ca + t * 128, 128)],
                        buf_v.at[q * 2 + t],
                        sem,
                    )
                )
        for c in copies:
            c.wait()
        lanes = lax.iota(jnp.int32, 16)
        for q in range(_RPW):
            rv = jnp.full((16,), rows16[q] & jnp.int32(7), jnp.int32)
            phv = jnp.full((16,), starts16[q] - cas[q], jnp.int32) + lanes
            for j0 in range(0, _COPY, 16):
                lc = phv + j0
                bv = (lc >> 7) + (q * 2)
                out_v[pl.ds(q * _COPY + j0, 16)] = plsc.load_gather(
                    buf_v, [bv, rv, lc & jnp.int32(127)]
                )
        pltpu.sync_copy(out_v, out_hbm.at[pl.ds(wid * (_RPW * _COPY), _RPW * _COPY)])

    return k(rpi, table, seq, tk1, st1)


def kernel(req_pool_indices, req_to_token, seq_lens, topk, speculative_num_steps):
    tk1 = jnp.reshape(jnp.asarray(topk, jnp.int32), (1,))
    st1 = jnp.reshape(jnp.asarray(speculative_num_steps, jnp.int32), (1,))
    return _sc_gather(
        req_pool_indices.astype(jnp.int32),
        req_to_token.astype(jnp.int32),
        seq_lens.astype(jnp.int32),
        tk1,
        st1,
    )


# speedup vs baseline: 3.4577x; 1.0587x over previous
import functools

import jax
import jax.numpy as jnp
from jax import lax
from jax.experimental import pallas as pl
from jax.experimental.pallas import tpu as pltpu
from jax.experimental.pallas import tpu_sc as plsc

_BS = 128
_COPY = 64
_NW = 16
_RPW = _BS // _NW
_POOL_LEN = 32768


def _sc_gather(rpi, table, seq, tk1, st1):
    mesh = plsc.VectorSubcoreMesh(
        core_axis_name="c", subcore_axis_name="s", num_cores=1
    )

    @functools.partial(
        pl.kernel,
        out_type=jax.ShapeDtypeStruct((_BS * _COPY,), jnp.int32),
        mesh=mesh,
        compiler_params=pltpu.CompilerParams(
            needs_layout_passes=False, skip_device_barrier=True
        ),
        scratch_types=[
            pltpu.VMEM((_BS + 16,), jnp.int32),
            pltpu.VMEM((_BS + 16,), jnp.int32),
            pltpu.VMEM((16,), jnp.int32),
            pltpu.VMEM((_RPW * 2, 8, 128), jnp.int32),
            pltpu.VMEM((_RPW * _COPY,), jnp.int32),
            pltpu.SemaphoreType.DMA,
        ],
    )
    def k(rpi_hbm, table_hbm, seq_hbm, tk_hbm, st_hbm, out_hbm,
          rpi_v, seq_v, sc_v, buf_v, out_v, sem):
        wid = lax.axis_index("s") + lax.axis_index("c") * 16
        base = wid * _RPW
        c1 = pltpu.async_copy(rpi_hbm, rpi_v.at[pl.ds(0, _BS)], sem)
        c2 = pltpu.async_copy(seq_hbm, seq_v.at[pl.ds(0, _BS)], sem)
        c3 = pltpu.async_copy(tk_hbm, sc_v.at[pl.ds(0, 1)], sem)
        c4 = pltpu.async_copy(st_hbm, sc_v.at[pl.ds(8, 1)], sem)
        c1.wait()
        c2.wait()
        c3.wait()
        c4.wait()
        scal = sc_v[pl.ds(0, 16)]
        dep = scal[0] * scal[8] - jnp.int32(_COPY)
        rows16 = rpi_v[pl.ds(base, 16)]
        starts16 = seq_v[pl.ds(base, 16)] + dep
        copies = []
        cas = []
        for q in range(_RPW):
            r8 = pl.multiple_of(rows16[q] & ~jnp.int32(7), 8)
            ca = jnp.minimum(
                starts16[q] & ~jnp.int32(127), jnp.int32(_POOL_LEN - 256)
            )
            ca = pl.multiple_of(ca, 128)
            cas.append(ca)
            for t in range(2):
                copies.append(
                    pltpu.async_copy(
                        table_hbm.at[pl.ds(r8, 8), pl.ds(ca + t * 128, 128)],
                        buf_v.at[q * 2 + t],
                        sem,
                    )
                )
        for c in copies:
            c.wait()
        lanes = lax.iota(jnp.int32, 16)
        for q in range(_RPW):
            rv = jnp.full((16,), rows16[q] & jnp.int32(7), jnp.int32)
            phv = jnp.full((16,), starts16[q] - cas[q], jnp.int32) + lanes
            for j0 in range(0, _COPY, 16):
                lc = phv + j0
                bv = (lc >> 7) + (q * 2)
                out_v[pl.ds(q * _COPY + j0, 16)] = plsc.load_gather(
                    buf_v, [bv, rv, lc & jnp.int32(127)]
                )
        pltpu.sync_copy(out_v, out_hbm.at[pl.ds(wid * (_RPW * _COPY), _RPW * _COPY)])

    return k(rpi, table, seq, tk1, st1)


def kernel(req_pool_indices, req_to_token, seq_lens, topk, speculative_num_steps):
    tk1 = jnp.reshape(jnp.asarray(topk, jnp.int32), (1,))
    st1 = jnp.reshape(jnp.asarray(speculative_num_steps, jnp.int32), (1,))
    return _sc_gather(
        req_pool_indices.astype(jnp.int32),
        req_to_token.astype(jnp.int32),
        seq_lens.astype(jnp.int32),
        tk1,
        st1,
    )


# speedup vs baseline: 3.4910x; 1.0097x over previous
import functools

import jax
import jax.numpy as jnp
from jax import lax
from jax.experimental import pallas as pl
from jax.experimental.pallas import tpu as pltpu
from jax.experimental.pallas import tpu_sc as plsc

_BS = 128
_COPY = 64
_NW = 16
_RPW = _BS // _NW
_POOL_LEN = 32768


def _sc_gather(rpi, table, seq, tk1, st1):
    mesh = plsc.VectorSubcoreMesh(
        core_axis_name="c", subcore_axis_name="s", num_cores=1
    )

    @functools.partial(
        pl.kernel,
        out_type=jax.ShapeDtypeStruct((_BS * _COPY,), jnp.int32),
        mesh=mesh,
        compiler_params=pltpu.CompilerParams(
            needs_layout_passes=False, skip_device_barrier=True
        ),
        scratch_types=[
            pltpu.VMEM((_BS + 16,), jnp.int32),
            pltpu.VMEM((_BS + 16,), jnp.int32),
            pltpu.VMEM((16,), jnp.int32),
            pltpu.VMEM((_RPW, 8, 256), jnp.int32),
            pltpu.VMEM((_RPW * _COPY,), jnp.int32),
            pltpu.SemaphoreType.DMA,
        ],
    )
    def k(rpi_hbm, table_hbm, seq_hbm, tk_hbm, st_hbm, out_hbm,
          rpi_v, seq_v, sc_v, buf_v, out_v, sem):
        wid = lax.axis_index("s") + lax.axis_index("c") * 16
        base = wid * _RPW
        c1 = pltpu.async_copy(rpi_hbm, rpi_v.at[pl.ds(0, _BS)], sem)
        c2 = pltpu.async_copy(seq_hbm, seq_v.at[pl.ds(0, _BS)], sem)
        c3 = pltpu.async_copy(tk_hbm, sc_v.at[pl.ds(0, 1)], sem)
        c4 = pltpu.async_copy(st_hbm, sc_v.at[pl.ds(8, 1)], sem)
        c1.wait()
        c2.wait()
        c3.wait()
        c4.wait()
        scal = sc_v[pl.ds(0, 16)]
        dep = scal[0] * scal[8] - jnp.int32(_COPY)
        rows16 = rpi_v[pl.ds(base, 16)]
        starts16 = seq_v[pl.ds(base, 16)] + dep
        copies = []
        cas = []
        for q in range(_RPW):
            r8 = pl.multiple_of(rows16[q] & ~jnp.int32(7), 8)
            ca = jnp.minimum(
                starts16[q] & ~jnp.int32(127), jnp.int32(_POOL_LEN - 256)
            )
            ca = pl.multiple_of(ca, 128)
            cas.append(ca)
            copies.append(
                pltpu.async_copy(
                    table_hbm.at[pl.ds(r8, 8), pl.ds(ca, 256)],
                    buf_v.at[q],
                    sem,
                )
            )
        for c in copies:
            c.wait()
        lanes = lax.iota(jnp.int32, 16)
        qv = jnp.zeros((16,), jnp.int32)
        for q in range(_RPW):
            rv = jnp.full((16,), rows16[q] & jnp.int32(7), jnp.int32)
            phv = jnp.full((16,), starts16[q] - cas[q], jnp.int32) + lanes
            for j0 in range(0, _COPY, 16):
                lc = phv + j0
                out_v[pl.ds(q * _COPY + j0, 16)] = plsc.load_gather(
                    buf_v, [qv + q, rv, lc]
                )
        pltpu.sync_copy(out_v, out_hbm.at[pl.ds(wid * (_RPW * _COPY), _RPW * _COPY)])

    return k(rpi, table, seq, tk1, st1)


def kernel(req_pool_indices, req_to_token, seq_lens, topk, speculative_num_steps):
    tk1 = jnp.reshape(jnp.asarray(topk, jnp.int32), (1,))
    st1 = jnp.reshape(jnp.asarray(speculative_num_steps, jnp.int32), (1,))
    return _sc_gather(
        req_pool_indices.astype(jnp.int32),
        req_to_token.astype(jnp.int32),
        seq_lens.astype(jnp.int32),
        tk1,
        st1,
    )
